# baseline (speedup 1.0000x reference)
import functools

import jax
import jax.numpy as jnp
from jax.experimental import pallas as pl
from jax.experimental.pallas import tpu as pltpu


def _mm_kernel(x_ref, w_ref, b_ref, o_ref):
    o_ref[...] = (
        jnp.dot(x_ref[...].astype(jnp.bfloat16), w_ref[...].astype(jnp.bfloat16),
                preferred_element_type=jnp.float32)
        + b_ref[...]
    )


def _mm(x, w, b, bm=512, bn=1024):
    m, k = x.shape
    k2, n = w.shape
    grid = (m // bm, n // bn)
    return pl.pallas_call(
        _mm_kernel,
        grid=grid,
        in_specs=[
            pl.BlockSpec((bm, k), lambda i, j: (i, 0)),
            pl.BlockSpec((k, bn), lambda i, j: (0, j)),
            pl.BlockSpec((bn,), lambda i, j: (j,)),
        ],
        out_specs=pl.BlockSpec((bm, bn), lambda i, j: (i, j)),
        out_shape=jax.ShapeDtypeStruct((m, n), jnp.float32),
    )(x, w, b)


def kernel(q, k, edge_index, edge_features, Wq, bq, Wk, bk, Wv, bv,
           Wqr, bqr, Wkr, bkr, Wvr, bvr, Wo, bo):
    B, L, D = q.shape
    H = 16
    DH = D // H
    N = B * L
    E = edge_index.shape[1]
    scale = 1.0 / jnp.sqrt(jnp.asarray(DH, dtype=jnp.float32))

    qf2 = q.reshape(-1, D)
    kf2 = k.reshape(-1, D)
    Qf = _mm(qf2, Wq, bq) * scale
    Qr = _mm(qf2, Wqr, bqr) * scale
    Kf = _mm(kf2, Wk, bk)
    Vf = _mm(kf2, Wv, bv)
    Kr = _mm(edge_features, Wkr, bkr)
    Vr = _mm(edge_features, Wvr, bvr)

    src = edge_index[0].astype(jnp.int32)
    dst = edge_index[1].astype(jnp.int32)

    qe = jnp.take(Qf, dst, axis=0).reshape(-1, H, DH)
    qre = jnp.take(Qr, dst, axis=0).reshape(-1, H, DH)
    ke = jnp.take(Kf, src, axis=0).reshape(-1, H, DH)
    ve = jnp.take(Vf, src, axis=0).reshape(-1, H, DH)
    kre = Kr.reshape(-1, H, DH)
    vre = Vr.reshape(-1, H, DH)

    score = (qe * ke).sum(-1) + (qre * kre).sum(-1)
    w = jnp.exp(score)
    denom = jax.ops.segment_sum(w, dst, num_segments=N)
    U = jax.ops.segment_sum(w[:, :, None] * (ve + vre), dst, num_segments=N)
    agg = U / (denom[:, :, None] + 1e-9)
    out = _mm(agg.reshape(N, D), Wo, bo)
    return out.reshape(B, L, D)


# speedup vs baseline: 2.2018x; 2.2018x over previous
import functools

import jax
import jax.numpy as jnp
from jax import lax
from jax.experimental import pallas as pl
from jax.experimental.pallas import tpu as pltpu, tpu_sc as plsc

H = 16
DH = 128
P = 18
CHUNK = 8
NC = 2
NS = 16


def _mm_kernel(x_ref, w_ref, b_ref, o_ref):
    o_ref[...] = (
        jnp.dot(x_ref[...].astype(jnp.bfloat16), w_ref[...].astype(jnp.bfloat16),
                preferred_element_type=jnp.float32)
        + b_ref[...]
    )


def _mm(x, w, b, bm=512, bn=1024):
    m, k = x.shape
    _, n = w.shape
    return pl.pallas_call(
        _mm_kernel,
        grid=(m // bm, n // bn),
        in_specs=[
            pl.BlockSpec((bm, k), lambda i, j: (i, 0)),
            pl.BlockSpec((k, bn), lambda i, j: (0, j)),
            pl.BlockSpec((bn,), lambda i, j: (j,)),
        ],
        out_specs=pl.BlockSpec((bm, bn), lambda i, j: (i, j)),
        out_shape=jax.ShapeDtypeStruct((m, n), jnp.float32),
    )(x, w, b)


def _sc_body(qtab, qrtab, ktab, vtab, krtab, vrtab, dst_hbm, src_hbm,
             chb, whb, dst8, src8, gq, gqr, gk, gv, gkr, gvr, contrib, wcon,
             s0, s1, s2, s3, s4, s5):
    cid = lax.axis_index("c")
    sid = lax.axis_index("s")
    wid = sid * NC + cid
    n_edges = dst_hbm.shape[0]
    ep = n_edges // (NC * NS)
    n_chunks = ep // CHUNK
    ebase = wid * ep
    D = qtab.shape[1]
    CWR = P * DH

    def chunk_body(ci, _):
        e0 = ebase + ci * CHUNK
        pltpu.sync_copy(dst_hbm.at[pl.ds(e0, CHUNK)], dst8)
        pltpu.sync_copy(src_hbm.at[pl.ds(e0, CHUNK)], src8)
        cpq = pltpu.async_copy(qtab.at[dst8], gq, s0)
        cpqr = pltpu.async_copy(qrtab.at[dst8], gqr, s1)
        cpk = pltpu.async_copy(ktab.at[src8], gk, s2)
        cpv = pltpu.async_copy(vtab.at[src8], gv, s3)
        cpkr = pltpu.async_copy(krtab.at[pl.ds(e0, CHUNK)], gkr, s4)
        cpvr = pltpu.async_copy(vrtab.at[pl.ds(e0, CHUNK)], gvr, s5)
        cpq.wait(); cpqr.wait(); cpk.wait(); cpv.wait(); cpkr.wait(); cpvr.wait()

        for e in range(CHUNK):
            def hbody(h, _):
                hb = h * DH
                acc = jnp.zeros((16,), jnp.float32)
                for j8 in range(DH // 16):
                    sl = pl.ds(hb + j8 * 16, 16)
                    acc = (acc + gq[e, sl] * gk[e, sl]
                           + gqr[e, sl] * gkr[e, sl])
                acc = acc + lax.rev(acc, (0,))
                s = ((acc[0] + acc[1]) + (acc[2] + acc[3])) + \
                    ((acc[4] + acc[5]) + (acc[6] + acc[7]))
                wv = jnp.exp(acc * 0.0 + s)
                for j8 in range(DH // 16):
                    sl = pl.ds(hb + j8 * 16, 16)
                    contrib[e, sl] = (gv[e, sl] + gvr[e, sl]) * wv
                wcon[e, pl.ds(h * 16, 16)] = wv
                return 0
            lax.fori_loop(0, H, hbody, 0)

        pltpu.sync_copy(contrib, chb.at[pl.ds(e0, CHUNK)])
        pltpu.sync_copy(wcon, whb.at[pl.ds(e0, CHUNK)])
        return 0

    lax.fori_loop(0, n_chunks, chunk_body, 0)


def _sc_edge(qtab, qrtab, ktab, vtab, krtab, vrtab, dst32, src32):
    E = dst32.shape[0]
    mesh = plsc.VectorSubcoreMesh(core_axis_name="c", subcore_axis_name="s",
                                  num_cores=NC, num_subcores=NS)
    D = qtab.shape[1]
    fn = pl.kernel(
        _sc_body,
        out_type=(jax.ShapeDtypeStruct((E, H * DH), jnp.float32),
                  jax.ShapeDtypeStruct((E, H * 16), jnp.float32)),
        mesh=mesh,
        scratch_types=[
            pltpu.VMEM((CHUNK,), jnp.int32),
            pltpu.VMEM((CHUNK,), jnp.int32),
            pltpu.VMEM((CHUNK, D), jnp.float32),
            pltpu.VMEM((CHUNK, D), jnp.float32),
            pltpu.VMEM((CHUNK, D), jnp.float32),
            pltpu.VMEM((CHUNK, D), jnp.float32),
            pltpu.VMEM((CHUNK, D), jnp.float32),
            pltpu.VMEM((CHUNK, D), jnp.float32),
            pltpu.VMEM((CHUNK, H * DH), jnp.float32),
            pltpu.VMEM((CHUNK, H * 16), jnp.float32),
            pltpu.SemaphoreType.DMA,
            pltpu.SemaphoreType.DMA,
            pltpu.SemaphoreType.DMA,
            pltpu.SemaphoreType.DMA,
            pltpu.SemaphoreType.DMA,
            pltpu.SemaphoreType.DMA,
        ],
    )
    return fn(qtab, qrtab, ktab, vtab, krtab, vrtab, dst32, src32)


def _acc_kernel(dst_ref, c_ref, o_ref):
    @pl.when(pl.program_id(1) == 0)
    def _():
        o_ref[...] = jnp.zeros_like(o_ref)

    eb = c_ref.shape[0]

    def ebody(e, _):
        d = dst_ref[e]
        o_ref[pl.ds(d, 1)] = o_ref[pl.ds(d, 1)] + c_ref[pl.ds(e, 1)]
        return 0
    lax.fori_loop(0, eb, ebody, 0)


def _segsum_u(contrib3, dst32, n_nodes, eb=512):
    e = contrib3.shape[0]
    hp = H // 2
    return pl.pallas_call(
        _acc_kernel,
        grid=(2, e // eb),
        in_specs=[
            pl.BlockSpec((eb,), lambda p, i: (i,), memory_space=pltpu.SMEM),
            pl.BlockSpec((eb, hp, DH), lambda p, i: (i, p, 0)),
        ],
        out_specs=pl.BlockSpec((n_nodes, hp, DH), lambda p, i: (0, p, 0)),
        out_shape=jax.ShapeDtypeStruct((n_nodes, H, DH), jnp.float32),
    )(dst32, contrib3)


def _accw_kernel(dst_ref, c_ref, o_ref):
    @pl.when(pl.program_id(0) == 0)
    def _():
        o_ref[...] = jnp.zeros_like(o_ref)

    eb = c_ref.shape[0]

    def ebody(e, _):
        d = dst_ref[e]
        o_ref[pl.ds(d, 1)] = o_ref[pl.ds(d, 1)] + c_ref[pl.ds(e, 1)]
        return 0
    lax.fori_loop(0, eb, ebody, 0)


def _segsum_w(wh, dst32, n_nodes, eb=512):
    e = wh.shape[0]
    cw = wh.shape[1]
    return pl.pallas_call(
        _accw_kernel,
        grid=(e // eb,),
        in_specs=[
            pl.BlockSpec((eb,), lambda i: (i,), memory_space=pltpu.SMEM),
            pl.BlockSpec((eb, cw), lambda i: (i, 0)),
        ],
        out_specs=pl.BlockSpec((n_nodes, cw), lambda i: (0, 0)),
        out_shape=jax.ShapeDtypeStruct((n_nodes, cw), jnp.float32),
    )(dst32, wh)


def _out_kernel(u_ref, w2_ref, wo_ref, bo_ref, o_ref):
    u = u_ref[...]
    w2 = w2_ref[...]
    acc = jnp.broadcast_to(bo_ref[...][None, :], o_ref.shape).astype(jnp.float32)
    for h in range(H):
        den = w2[:, h * 16:h * 16 + 1] + 1e-9
        agg = u[:, h, :] / den
        acc = acc + jnp.dot(agg.astype(jnp.bfloat16),
                            wo_ref[0, h].astype(jnp.bfloat16),
                            preferred_element_type=jnp.float32)
    o_ref[...] = acc


def _out_proj(u3, w2, Wo, bo, bm=256):
    n = u3.shape[0]
    d = Wo.shape[1]
    wo3 = Wo.reshape(1, H, DH, d)
    return pl.pallas_call(
        _out_kernel,
        grid=(n // bm,),
        in_specs=[
            pl.BlockSpec((bm, H, DH), lambda i: (i, 0, 0)),
            pl.BlockSpec((bm, H * 16), lambda i: (i, 0)),
            pl.BlockSpec((1, H, DH, d), lambda i: (0, 0, 0, 0)),
            pl.BlockSpec((d,), lambda i: (0,)),
        ],
        out_specs=pl.BlockSpec((bm, d), lambda i: (i, 0)),
        out_shape=jax.ShapeDtypeStruct((n, d), jnp.float32),
    )(u3, w2, wo3, bo)


def kernel(q, k, edge_index, edge_features, Wq, bq, Wk, bk, Wv, bv,
           Wqr, bqr, Wkr, bkr, Wvr, bvr, Wo, bo):
    B, L, D = q.shape
    N = B * L
    E = edge_index.shape[1]
    scale = 1.0 / jnp.sqrt(jnp.asarray(DH, dtype=jnp.float32))

    q2 = q.reshape(N, D)
    k2 = k.reshape(N, D)
    Qf = _mm(q2, Wq * scale, bq * scale)
    Qr = _mm(q2, Wqr * scale, bqr * scale)
    Kf = _mm(k2, Wk, bk)
    Vf = _mm(k2, Wv, bv)
    Kr = _mm(edge_features, Wkr, bkr)
    Vr = _mm(edge_features, Wvr, bvr)

    src32 = edge_index[0].astype(jnp.int32)
    dst32 = edge_index[1].astype(jnp.int32)

    ch, wh = _sc_edge(Qf, Qr, Kf, Vf, Kr, Vr, dst32, src32)
    u3 = _segsum_u(ch.reshape(E, H, DH), dst32, N)
    w2 = _segsum_w(wh, dst32, N)
    out = _out_proj(u3, w2, Wo, bo)
    return out.reshape(B, L, D)
